# baseline (speedup 1.0000x reference)
import functools

import jax
import jax.numpy as jnp
from jax import lax
from jax.experimental import pallas as pl
from jax.experimental.pallas import tpu as pltpu
from jax.experimental.pallas import tpu_sc as plsc

N = 10000
F_IN = 128
HID = 64
HEADS = 8
PROJ = 32
E = 160000
K = 50

NPAD = 10240
BR = 256
NBLK = NPAD // BR

EPAD = 163840
EBLK = 8192
NEB = EPAD // EBLK

NC, NS = 2, 16
NW = NC * NS
WIN = 128

_SC_PARAMS = pltpu.CompilerParams(use_tc_tiling_on_sc=False)

_HI = jax.lax.Precision.HIGHEST


def _bf16dot(a, b):
    return lax.dot_general(a.astype(jnp.bfloat16), b.astype(jnp.bfloat16),
                           (((1,), (0,)), ((), ())),
                           preferred_element_type=jnp.float32)


def _dense1_body(x_ref, w1_ref, b1_ref, wg1_ref, as1_ref, ad1_ref,
                 x1_ref, h_ref, es_ref, ed_ref):
    xb = x_ref[...]
    x1_ref[...] = jnp.maximum(_bf16dot(xb, w1_ref[...]) + b1_ref[...], 0.0)
    z = jnp.zeros((BR, 8), jnp.float32)
    es_ref[:, 8:16] = z
    ed_ref[:, 8:16] = z
    for k in range(HEADS):
        hk = _bf16dot(xb, wg1_ref[:, k * HID:(k + 1) * HID])
        h_ref[k] = hk
        es_ref[:, k:k + 1] = jnp.sum(hk * as1_ref[k:k + 1, :], axis=1,
                                     keepdims=True)
        ed_ref[:, k:k + 1] = jnp.sum(hk * ad1_ref[k:k + 1, :], axis=1,
                                     keepdims=True)


def _make_gw():
    per_w = EPAD // NW
    nwin = per_w // WIN
    mesh = plsc.VectorSubcoreMesh(core_axis_name="c", subcore_axis_name="s")
    assert nwin % 2 == 0 and nwin >= 4

    @functools.partial(
        pl.kernel, mesh=mesh,
        out_type=[jax.ShapeDtypeStruct((EPAD, 16), jnp.float32),
                  jax.ShapeDtypeStruct((NC, NPAD, 16), jnp.float32)],
        scratch_types=[pltpu.VMEM((2, WIN), jnp.int32),
                       pltpu.VMEM((2, WIN), jnp.int32),
                       pltpu.VMEM((2, WIN, 16), jnp.float32),
                       pltpu.VMEM((2, WIN, 16), jnp.float32),
                       pltpu.VMEM((WIN, 16), jnp.float32),
                       pltpu.VMEM((WIN, 16), jnp.float32),
                       pltpu.SemaphoreType.DMA,
                       pltpu.SemaphoreType.DMA,
                       pltpu.SemaphoreType.DMA,
                       pltpu.SemaphoreType.DMA,
                       pltpu.VMEM_SHARED((NPAD, 16), jnp.float32)],
        compiler_params=_SC_PARAMS,
    )
    def gw(es_hbm, ed_hbm, src_hbm, dst_hbm, w_hbm, den_hbm,
           idxs_v, idxd_v, ra_v, rb_v, w_v, zero_v, sa0, sa1, sb0, sb1,
           den_sh):
        core = lax.axis_index("c")
        sid = lax.axis_index("s")
        ebase = (core * NS + sid) * per_w
        sems = ((sa0, sb0), (sa1, sb1))

        @pl.loop(0, WIN)
        def _(j):
            zero_v[j, pl.ds(0, 16)] = jnp.zeros((16,), jnp.float32)

        for t in range(NPAD // NS // WIN):
            pltpu.sync_copy(
                zero_v, den_sh.at[pl.ds(sid * (NPAD // NS) + t * WIN, WIN)])
        plsc.subcore_barrier()

        def prefetch(n, b):
            base = ebase + n * WIN
            pltpu.sync_copy(src_hbm.at[pl.ds(base, WIN)], idxs_v.at[b])
            pltpu.sync_copy(dst_hbm.at[pl.ds(base, WIN)], idxd_v.at[b])
            pltpu.make_async_copy(es_hbm.at[idxs_v.at[b]], ra_v.at[b],
                                  sems[b][0]).start()
            pltpu.make_async_copy(ed_hbm.at[idxd_v.at[b]], rb_v.at[b],
                                  sems[b][1]).start()

        def consume(n, b):
            base = ebase + n * WIN
            pltpu.make_async_copy(es_hbm.at[idxs_v.at[b]], ra_v.at[b],
                                  sems[b][0]).wait()
            pltpu.make_async_copy(ed_hbm.at[idxd_v.at[b]], rb_v.at[b],
                                  sems[b][1]).wait()

            @pl.loop(0, WIN)
            def _(j):
                e = ra_v[b, j, pl.ds(0, 16)] + rb_v[b, j, pl.ds(0, 16)]
                e = jnp.where(e > 0, e, 0.2 * e)
                w = jnp.exp(e)
                w = jnp.where(base + j < E, w, jnp.zeros((16,), jnp.float32))
                w_v[j, pl.ds(0, 16)] = w

            pltpu.sync_copy(w_v, w_hbm.at[pl.ds(base, WIN)])
            pltpu.sync_copy(w_v, den_sh.at[idxd_v.at[b]], add=True)

        prefetch(0, 0)
        prefetch(1, 1)

        @pl.loop(0, nwin // 2 - 1)
        def _(p):
            n = p * 2
            consume(n, 0)
            prefetch(n + 2, 0)
            consume(n + 1, 1)
            prefetch(n + 3, 1)

        consume(nwin - 2, 0)
        consume(nwin - 1, 1)

        plsc.subcore_barrier()
        pltpu.sync_copy(
            den_sh.at[pl.ds(sid * (NPAD // NS), NPAD // NS)],
            den_hbm.at[core, pl.ds(sid * (NPAD // NS), NPAD // NS)])
        plsc.subcore_barrier()

    return gw


def _w1_body(w16_ref, wt_ref):
    wt_ref[...] = w16_ref[:, 0:HEADS].T


def _w2_body(w16_ref, wt_ref):
    wt_ref[...] = w16_ref[:, 0:1].T


def _make_agg(pairs_per_core, edges_per_pass, width):
    per_w = edges_per_pass // NS
    nwin = per_w // WIN
    npass = pairs_per_core
    nslot = npass * NC
    nh = width // HID
    mesh = plsc.VectorSubcoreMesh(core_axis_name="c", subcore_axis_name="s")
    assert nwin % 2 == 0 and nwin >= 4

    @functools.partial(
        pl.kernel, mesh=mesh,
        out_type=jax.ShapeDtypeStruct((nslot, NPAD, width), jnp.float32),
        scratch_types=[pltpu.VMEM((2, WIN), jnp.int32),
                       pltpu.VMEM((2, WIN), jnp.int32),
                       pltpu.VMEM((2, WIN), jnp.int32),
                       pltpu.VMEM((2, WIN), jnp.int32),
                       pltpu.VMEM((2, 2, WIN), jnp.float32),
                       pltpu.VMEM((2, 2, WIN, HID), jnp.float32),
                       pltpu.VMEM((2, WIN, width), jnp.float32),
                       pltpu.VMEM((WIN, width), jnp.float32),
                       pltpu.SemaphoreType.DMA,
                       pltpu.SemaphoreType.DMA,
                       pltpu.SemaphoreType.DMA,
                       pltpu.SemaphoreType.DMA,
                       pltpu.SemaphoreType.DMA,
                       pltpu.SemaphoreType.DMA,
                       pltpu.VMEM_SHARED((NPAD, width), jnp.float32)],
        compiler_params=_SC_PARAMS,
    )
    def agg(hflat_hbm, src_hbm, dst_hbm, wt_hbm, uout_hbm,
            idx0_v, idx1_v, dst_v, dsts_v, wv, rows_v, prod_v, zero_v,
            s00, s01, s10, s11, sp0, sp1, acc_sh):
        core = lax.axis_index("c")
        sid = lax.axis_index("s")
        sems = ((s00, s01), (s10, s11))
        psems = (sp0, sp1)

        @pl.loop(0, WIN)
        def _(j):
            for q in range(width // 16):
                zero_v[j, pl.ds(q * 16, 16)] = jnp.zeros((16,), jnp.float32)

        @pl.loop(0, npass)
        def _(kk):
            if npass > 1:
                k0 = (npass * core + kk) * nh
                slot = npass * core + kk
                ebase = sid * per_w
            else:
                k0 = 0
                slot = core
                ebase = (core * NS + sid) * per_w

            for t in range(NPAD // NS // WIN):
                pltpu.sync_copy(
                    zero_v, acc_sh.at[pl.ds(sid * (NPAD // NS) + t * WIN,
                                            WIN)])
            plsc.subcore_barrier()

            def prefetch(n, b):
                base = ebase + n * WIN
                pltpu.sync_copy(src_hbm.at[pl.ds(base, WIN)], idx0_v.at[b])
                pltpu.sync_copy(dst_hbm.at[pl.ds(base, WIN)], dst_v.at[b])
                pltpu.sync_copy(wt_hbm.at[k0, pl.ds(base, WIN)],
                                wv.at[b, 0])
                if nh > 1:
                    pltpu.sync_copy(wt_hbm.at[k0 + 1, pl.ds(base, WIN)],
                                    wv.at[b, 1])
                for q in range(WIN // 16):
                    sl = pl.ds(q * 16, 16)
                    if nh > 1:
                        idx1_v[b, sl] = idx0_v[b, sl] + (k0 + 1) * NPAD
                    if npass > 1:
                        idx0_v[b, sl] = idx0_v[b, sl] + k0 * NPAD
                pltpu.make_async_copy(hflat_hbm.at[idx0_v.at[b]],
                                      rows_v.at[b, 0], sems[b][0]).start()
                if nh > 1:
                    pltpu.make_async_copy(hflat_hbm.at[idx1_v.at[b]],
                                          rows_v.at[b, 1], sems[b][1]).start()

            def add_copy(b):
                return pltpu.make_async_copy(
                    prod_v.at[b], acc_sh.at[dsts_v.at[b]], psems[b])

            def consume(b):
                pltpu.make_async_copy(hflat_hbm.at[idx0_v.at[b]],
                                      rows_v.at[b, 0], sems[b][0]).wait()
                if nh > 1:
                    pltpu.make_async_copy(hflat_hbm.at[idx1_v.at[b]],
                                          rows_v.at[b, 1], sems[b][1]).wait()
                add_copy(b).wait()
                for q in range(WIN // 16):
                    sl = pl.ds(q * 16, 16)
                    dsts_v[b, sl] = dst_v[b, sl]

                @pl.loop(0, WIN, step=16)
                def _(j0):
                    for jj in range(16):
                        for i in range(nh):
                            splat = jnp.full((16,), 1.0, jnp.float32) * (
                                wv[b, i, pl.ds(j0, 16)][jj])
                            for q in range(HID // 16):
                                prod_v[b, j0 + jj,
                                       pl.ds(i * HID + q * 16, 16)] = (
                                    rows_v[b, i, j0 + jj, pl.ds(q * 16, 16)]
                                    * splat)

                add_copy(b).start(add=True)

            prefetch(0, 0)
            prefetch(1, 1)
            for bb in range(2):
                for q in range(WIN // 16):
                    sl = pl.ds(q * 16, 16)
                    dsts_v[bb, sl] = dst_v[bb, sl]
            pltpu.make_async_copy(zero_v, acc_sh.at[dsts_v.at[0]],
                                  psems[0]).start(add=True)
            pltpu.make_async_copy(zero_v, acc_sh.at[dsts_v.at[1]],
                                  psems[1]).start(add=True)

            @pl.loop(0, nwin // 2 - 1)
            def _(p):
                n = p * 2
                consume(0)
                prefetch(n + 2, 0)
                consume(1)
                prefetch(n + 3, 1)

            consume(0)
            consume(1)
            add_copy(0).wait()
            add_copy(1).wait()

            plsc.subcore_barrier()
            pltpu.sync_copy(
                acc_sh.at[pl.ds(sid * (NPAD // NS), NPAD // NS)],
                uout_hbm.at[slot, pl.ds(sid * (NPAD // NS), NPAD // NS)])
            plsc.subcore_barrier()

    return agg


def _dense2_body(uout_ref, d0_ref, d1_ref, bg1_ref, wg2_ref, as2_ref,
                 ad2_ref, h2_ref, es2_ref, ed2_ref):
    den16 = d0_ref[0] + d1_ref[0] + 1e-16
    parts = []
    for k in range(HEADS):
        num = uout_ref[k]
        den = den16[:, k:k + 1]
        parts.append(jnp.maximum(num / den + bg1_ref[:, k * HID:(k + 1) * HID],
                                 0.0))
    xg = jnp.concatenate(parts, axis=1)
    h2 = _bf16dot(xg, wg2_ref[...])
    h2_ref[...] = h2
    z = jnp.zeros((BR, 15), jnp.float32)
    es2 = jnp.sum(h2 * as2_ref[...], axis=1, keepdims=True)
    ed2 = jnp.sum(h2 * ad2_ref[...], axis=1, keepdims=True)
    es2_ref[...] = jnp.concatenate([es2, z], axis=1)
    ed2_ref[...] = jnp.concatenate([ed2, z], axis=1)


def _final_body(x1_ref, u0_ref, u1_ref, d0_ref, d1_ref, bg2_ref, wp_ref,
                bp_ref, final_ref, g_ref, gt_ref, sq_ref):
    i = pl.program_id(0)
    num = u0_ref[0] + u1_ref[0]
    den = d0_ref[0][:, 0:1] + d1_ref[0][:, 0:1] + 1e-16
    x2 = num / den + bg2_ref[...]
    fb = jnp.concatenate([x1_ref[...], x2], axis=1)
    final_ref[...] = fb
    g = _bf16dot(fb, wp_ref[...]) + bp_ref[...]
    sq = jnp.sum(g * g, axis=1, keepdims=True)
    row = i * BR + lax.broadcasted_iota(jnp.int32, (BR, 1), 0)
    sq_ref[...] = jnp.where(row < N, sq, 1e30)
    g_ref[...] = g
    gt_ref[...] = g.T


def _knn_body(g_ref, gt_ref, sq_ref, sqrow_ref, final_ref, fblk_ref,
              gln_ref, bln_ref, wo1_ref, wo2_ref, bo_ref,
              out_ref, key_ref, key16_ref, mask_ref):
    dots = _bf16dot(g_ref[...], gt_ref[...])
    d = jnp.maximum(sq_ref[...] + sqrow_ref[...] - 2.0 * dots, 0.0)
    key = lax.bitcast_convert_type(d, jnp.int32)
    key_ref[...] = key
    key16_ref[...] = (key >> 16).astype(jnp.int16)

    lo0 = jnp.zeros((BR, 1), jnp.int32)
    hi0 = jnp.full((BR, 1), 0x7F80, jnp.int32)

    def bs16_step(_, carry):
        lo, hi = carry
        mid = lo + ((hi - lo) >> 1)
        cnt = jnp.sum((key16_ref[...] <= mid.astype(jnp.int16))
                      .astype(jnp.int16), axis=1, keepdims=True,
                      dtype=jnp.int32)
        pred = cnt >= K
        return jnp.where(pred, lo, mid + 1), jnp.where(pred, mid, hi)

    _, t16 = lax.fori_loop(0, 15, bs16_step, (lo0, hi0))

    lo0 = t16 << 16
    hi0 = (t16 << 16) | 0xFFFF

    def bs_step(_, carry):
        lo, hi = carry
        mid = lo + ((hi - lo) >> 1)
        cnt = jnp.sum((key_ref[...] <= mid).astype(jnp.int32), axis=1,
                      keepdims=True)
        pred = cnt >= K
        return jnp.where(pred, lo, mid + 1), jnp.where(pred, mid, hi)

    lo, hi = lax.fori_loop(0, 16, bs_step, (lo0, hi0))

    mask = (key_ref[...] <= hi).astype(jnp.float32)
    cnt = jnp.sum(mask, axis=1, keepdims=True)
    mask_ref[...] = mask

    sim = lax.dot_general(mask_ref[...], final_ref[...],
                          (((1,), (0,)), ((), ())), precision=_HI) / cnt
    mu = jnp.mean(sim, axis=1, keepdims=True)
    var = jnp.mean((sim - mu) ** 2, axis=1, keepdims=True)
    simn = (sim - mu) * lax.rsqrt(var + 1e-5) * gln_ref[...] + bln_ref[...]

    out = (lax.dot_general(fblk_ref[...], wo1_ref[...],
                           (((1,), (0,)), ((), ())), precision=_HI)
           + lax.dot_general(simn, wo2_ref[...],
                             (((1,), (0,)), ((), ())), precision=_HI)
           + bo_ref[...])
    out_ref[...] = out


_gw = _make_gw()
_agg1 = _make_agg(4, EPAD, HID)
_agg2 = _make_agg(1, EPAD // 2, HID)


def _full(x, edge_index, W1, b1, Wg1, as1, ad1, bg1, Wg2, as2, ad2, bg2,
          Wp, bp, g_ln, b_ln, Wo, bo):
    xpad = jnp.zeros((NPAD, F_IN), jnp.float32).at[:N].set(x)
    srcp = jnp.zeros((EPAD,), jnp.int32).at[:E].set(edge_index[0])
    dstp = jnp.zeros((EPAD,), jnp.int32).at[:E].set(edge_index[1])

    x1, h, es16, ed16 = pl.pallas_call(
        _dense1_body,
        grid=(NBLK,),
        in_specs=[
            pl.BlockSpec((BR, F_IN), lambda i: (i, 0)),
            pl.BlockSpec((F_IN, HID), lambda i: (0, 0)),
            pl.BlockSpec((1, HID), lambda i: (0, 0)),
            pl.BlockSpec((F_IN, HEADS * HID), lambda i: (0, 0)),
            pl.BlockSpec((HEADS, HID), lambda i: (0, 0)),
            pl.BlockSpec((HEADS, HID), lambda i: (0, 0)),
        ],
        out_specs=[
            pl.BlockSpec((BR, HID), lambda i: (i, 0)),
            pl.BlockSpec((HEADS, BR, HID), lambda i: (0, i, 0)),
            pl.BlockSpec((BR, 16), lambda i: (i, 0)),
            pl.BlockSpec((BR, 16), lambda i: (i, 0)),
        ],
        out_shape=[
            jax.ShapeDtypeStruct((NPAD, HID), jnp.float32),
            jax.ShapeDtypeStruct((HEADS, NPAD, HID), jnp.float32),
            jax.ShapeDtypeStruct((NPAD, 16), jnp.float32),
            jax.ShapeDtypeStruct((NPAD, 16), jnp.float32),
        ],
    )(xpad, W1, b1.reshape(1, HID), Wg1, as1, ad1)

    w16_1, den1 = _gw(es16, ed16, srcp, dstp)

    wt1 = pl.pallas_call(
        _w1_body,
        grid=(NEB,),
        in_specs=[pl.BlockSpec((EBLK, 16), lambda i: (i, 0))],
        out_specs=pl.BlockSpec((HEADS, EBLK), lambda i: (0, i)),
        out_shape=jax.ShapeDtypeStruct((HEADS, EPAD), jnp.float32),
    )(w16_1)

    hflat = h.reshape(HEADS * NPAD, HID)
    uout1 = _agg1(hflat, srcp, dstp, wt1)

    h2, es2, ed2 = pl.pallas_call(
        _dense2_body,
        grid=(NBLK,),
        in_specs=[
            pl.BlockSpec((HEADS, BR, HID), lambda i: (0, i, 0)),
            pl.BlockSpec((1, BR, 16), lambda i: (0, i, 0)),
            pl.BlockSpec((1, BR, 16), lambda i: (1, i, 0)),
            pl.BlockSpec((1, HEADS * HID), lambda i: (0, 0)),
            pl.BlockSpec((HEADS * HID, HID), lambda i: (0, 0)),
            pl.BlockSpec((1, HID), lambda i: (0, 0)),
            pl.BlockSpec((1, HID), lambda i: (0, 0)),
        ],
        out_specs=[pl.BlockSpec((BR, HID), lambda i: (i, 0)),
                   pl.BlockSpec((BR, 16), lambda i: (i, 0)),
                   pl.BlockSpec((BR, 16), lambda i: (i, 0))],
        out_shape=[jax.ShapeDtypeStruct((NPAD, HID), jnp.float32),
                   jax.ShapeDtypeStruct((NPAD, 16), jnp.float32),
                   jax.ShapeDtypeStruct((NPAD, 16), jnp.float32)],
    )(uout1, den1, den1, bg1.reshape(1, -1), Wg2, as2, ad2)

    w16_2, den2 = _gw(es2, ed2, srcp, dstp)

    wt2 = pl.pallas_call(
        _w2_body,
        grid=(NEB,),
        in_specs=[pl.BlockSpec((EBLK, 16), lambda i: (i, 0))],
        out_specs=pl.BlockSpec((1, EBLK), lambda i: (0, i)),
        out_shape=jax.ShapeDtypeStruct((1, EPAD), jnp.float32),
    )(w16_2)

    uout2 = _agg2(h2, srcp, dstp, wt2)

    final, g, gt, sq = pl.pallas_call(
        _final_body,
        grid=(NBLK,),
        in_specs=[
            pl.BlockSpec((BR, HID), lambda i: (i, 0)),
            pl.BlockSpec((1, BR, HID), lambda i: (0, i, 0)),
            pl.BlockSpec((1, BR, HID), lambda i: (1, i, 0)),
            pl.BlockSpec((1, BR, 16), lambda i: (0, i, 0)),
            pl.BlockSpec((1, BR, 16), lambda i: (1, i, 0)),
            pl.BlockSpec((1, HID), lambda i: (0, 0)),
            pl.BlockSpec((2 * HID, PROJ), lambda i: (0, 0)),
            pl.BlockSpec((1, PROJ), lambda i: (0, 0)),
        ],
        out_specs=[
            pl.BlockSpec((BR, 2 * HID), lambda i: (i, 0)),
            pl.BlockSpec((BR, PROJ), lambda i: (i, 0)),
            pl.BlockSpec((PROJ, BR), lambda i: (0, i)),
            pl.BlockSpec((BR, 1), lambda i: (i, 0)),
        ],
        out_shape=[
            jax.ShapeDtypeStruct((NPAD, 2 * HID), jnp.float32),
            jax.ShapeDtypeStruct((NPAD, PROJ), jnp.float32),
            jax.ShapeDtypeStruct((PROJ, NPAD), jnp.float32),
            jax.ShapeDtypeStruct((NPAD, 1), jnp.float32),
        ],
    )(x1, uout2, uout2, den2, den2, bg2.reshape(1, HID), Wp,
      bp.reshape(1, PROJ))

    sqrow = sq.reshape(1, NPAD)

    out = pl.pallas_call(
        _knn_body,
        grid=(NBLK,),
        in_specs=[
            pl.BlockSpec((BR, PROJ), lambda i: (i, 0)),
            pl.BlockSpec((PROJ, NPAD), lambda i: (0, 0)),
            pl.BlockSpec((BR, 1), lambda i: (i, 0)),
            pl.BlockSpec((1, NPAD), lambda i: (0, 0)),
            pl.BlockSpec((NPAD, 2 * HID), lambda i: (0, 0)),
            pl.BlockSpec((BR, 2 * HID), lambda i: (i, 0)),
            pl.BlockSpec((1, 2 * HID), lambda i: (0, 0)),
            pl.BlockSpec((1, 2 * HID), lambda i: (0, 0)),
            pl.BlockSpec((2 * HID, HID), lambda i: (0, 0)),
            pl.BlockSpec((2 * HID, HID), lambda i: (0, 0)),
            pl.BlockSpec((1, HID), lambda i: (0, 0)),
        ],
        out_specs=pl.BlockSpec((BR, HID), lambda i: (i, 0)),
        out_shape=jax.ShapeDtypeStruct((NPAD, HID), jnp.float32),
        scratch_shapes=[
            pltpu.VMEM((BR, NPAD), jnp.int32),
            pltpu.VMEM((BR, NPAD), jnp.int16),
            pltpu.VMEM((BR, NPAD), jnp.float32),
        ],
    )(g, gt, sq, sqrow, final, final, g_ln.reshape(1, -1),
      b_ln.reshape(1, -1), Wo[:2 * HID], Wo[2 * HID:], bo.reshape(1, HID))

    return out[:N]


def kernel(x, edge_index, W1, b1, Wg1, as1, ad1, bg1, Wg2, as2, ad2, bg2,
           Wp, bp, g_ln, b_ln, Wo, bo):
    return _full(x, edge_index, W1, b1, Wg1, as1, ad1, bg1, Wg2, as2, ad2,
                 bg2, Wp, bp, g_ln, b_ln, Wo, bo)


# speedup vs baseline: 1.1148x; 1.1148x over previous
import functools

import jax
import jax.numpy as jnp
from jax import lax
from jax.experimental import pallas as pl
from jax.experimental.pallas import tpu as pltpu
from jax.experimental.pallas import tpu_sc as plsc

N = 10000
F_IN = 128
HID = 64
HEADS = 8
PROJ = 32
E = 160000
K = 50

NPAD = 10240
BR = 256
NBLK = NPAD // BR

EPAD = 163840
EBLK = 8192
NEB = EPAD // EBLK

NC, NS = 2, 16
NW = NC * NS
WIN = 128

_SC_PARAMS = pltpu.CompilerParams(use_tc_tiling_on_sc=False)

_HI = jax.lax.Precision.HIGHEST


def _bf16dot(a, b):
    return lax.dot_general(a.astype(jnp.bfloat16), b.astype(jnp.bfloat16),
                           (((1,), (0,)), ((), ())),
                           preferred_element_type=jnp.float32)


def _dense1_body(x_ref, w1_ref, b1_ref, wg1_ref, as1_ref, ad1_ref,
                 x1_ref, h_ref, es_ref, ed_ref):
    xb = x_ref[...]
    x1_ref[...] = jnp.maximum(_bf16dot(xb, w1_ref[...]) + b1_ref[...], 0.0)
    z = jnp.zeros((BR, 8), jnp.float32)
    es_ref[:, 8:16] = z
    ed_ref[:, 8:16] = z
    for k in range(HEADS):
        hk = _bf16dot(xb, wg1_ref[:, k * HID:(k + 1) * HID])
        h_ref[k] = hk
        es_ref[:, k:k + 1] = jnp.sum(hk * as1_ref[k:k + 1, :], axis=1,
                                     keepdims=True)
        ed_ref[:, k:k + 1] = jnp.sum(hk * ad1_ref[k:k + 1, :], axis=1,
                                     keepdims=True)


def _make_gw():
    per_w = EPAD // NW
    nwin = per_w // WIN
    mesh = plsc.VectorSubcoreMesh(core_axis_name="c", subcore_axis_name="s")
    assert nwin % 2 == 0 and nwin >= 4

    @functools.partial(
        pl.kernel, mesh=mesh,
        out_type=[jax.ShapeDtypeStruct((EPAD, 16), jnp.float32),
                  jax.ShapeDtypeStruct((NC, NPAD, 16), jnp.float32)],
        scratch_types=[pltpu.VMEM((2, WIN), jnp.int32),
                       pltpu.VMEM((2, WIN), jnp.int32),
                       pltpu.VMEM((2, WIN, 16), jnp.float32),
                       pltpu.VMEM((2, WIN, 16), jnp.float32),
                       pltpu.VMEM((WIN, 16), jnp.float32),
                       pltpu.VMEM((WIN, 16), jnp.float32),
                       pltpu.SemaphoreType.DMA,
                       pltpu.SemaphoreType.DMA,
                       pltpu.SemaphoreType.DMA,
                       pltpu.SemaphoreType.DMA,
                       pltpu.VMEM_SHARED((NPAD, 16), jnp.float32)],
        compiler_params=_SC_PARAMS,
    )
    def gw(es_hbm, ed_hbm, src_hbm, dst_hbm, w_hbm, den_hbm,
           idxs_v, idxd_v, ra_v, rb_v, w_v, zero_v, sa0, sa1, sb0, sb1,
           den_sh):
        core = lax.axis_index("c")
        sid = lax.axis_index("s")
        ebase = (core * NS + sid) * per_w
        sems = ((sa0, sb0), (sa1, sb1))

        @pl.loop(0, WIN)
        def _(j):
            zero_v[j, pl.ds(0, 16)] = jnp.zeros((16,), jnp.float32)

        for t in range(NPAD // NS // WIN):
            pltpu.sync_copy(
                zero_v, den_sh.at[pl.ds(sid * (NPAD // NS) + t * WIN, WIN)])
        plsc.subcore_barrier()

        def prefetch(n, b):
            base = ebase + n * WIN
            pltpu.sync_copy(src_hbm.at[pl.ds(base, WIN)], idxs_v.at[b])
            pltpu.sync_copy(dst_hbm.at[pl.ds(base, WIN)], idxd_v.at[b])
            pltpu.make_async_copy(es_hbm.at[idxs_v.at[b]], ra_v.at[b],
                                  sems[b][0]).start()
            pltpu.make_async_copy(ed_hbm.at[idxd_v.at[b]], rb_v.at[b],
                                  sems[b][1]).start()

        def consume(n, b):
            base = ebase + n * WIN
            pltpu.make_async_copy(es_hbm.at[idxs_v.at[b]], ra_v.at[b],
                                  sems[b][0]).wait()
            pltpu.make_async_copy(ed_hbm.at[idxd_v.at[b]], rb_v.at[b],
                                  sems[b][1]).wait()

            @pl.loop(0, WIN)
            def _(j):
                e = ra_v[b, j, pl.ds(0, 16)] + rb_v[b, j, pl.ds(0, 16)]
                e = jnp.where(e > 0, e, 0.2 * e)
                w = jnp.exp(e)
                w = jnp.where(base + j < E, w, jnp.zeros((16,), jnp.float32))
                w_v[j, pl.ds(0, 16)] = w

            pltpu.sync_copy(w_v, w_hbm.at[pl.ds(base, WIN)])
            pltpu.sync_copy(w_v, den_sh.at[idxd_v.at[b]], add=True)

        prefetch(0, 0)
        prefetch(1, 1)

        @pl.loop(0, nwin // 2 - 1)
        def _(p):
            n = p * 2
            consume(n, 0)
            prefetch(n + 2, 0)
            consume(n + 1, 1)
            prefetch(n + 3, 1)

        consume(nwin - 2, 0)
        consume(nwin - 1, 1)

        plsc.subcore_barrier()
        pltpu.sync_copy(
            den_sh.at[pl.ds(sid * (NPAD // NS), NPAD // NS)],
            den_hbm.at[core, pl.ds(sid * (NPAD // NS), NPAD // NS)])
        plsc.subcore_barrier()

    return gw


def _w1_body(w16_ref, wt_ref):
    wt_ref[...] = w16_ref[:, 0:HEADS].T


def _w2_body(w16_ref, wt_ref):
    wt_ref[...] = w16_ref[:, 0:1].T


def _make_agg(pairs_per_core, edges_per_pass, width):
    per_w = edges_per_pass // NS
    nwin = per_w // WIN
    npass = pairs_per_core
    nslot = npass * NC
    nh = width // HID
    mesh = plsc.VectorSubcoreMesh(core_axis_name="c", subcore_axis_name="s")
    assert nwin % 2 == 0 and nwin >= 4

    @functools.partial(
        pl.kernel, mesh=mesh,
        out_type=jax.ShapeDtypeStruct((nslot, NPAD, width), jnp.float32),
        scratch_types=[pltpu.VMEM((2, WIN), jnp.int32),
                       pltpu.VMEM((2, WIN), jnp.int32),
                       pltpu.VMEM((2, WIN), jnp.int32),
                       pltpu.VMEM((2, WIN), jnp.int32),
                       pltpu.VMEM((2, 2, WIN), jnp.float32),
                       pltpu.VMEM((2, 2, WIN, HID), jnp.float32),
                       pltpu.VMEM((2, WIN, width), jnp.float32),
                       pltpu.VMEM((WIN, width), jnp.float32),
                       pltpu.SemaphoreType.DMA,
                       pltpu.SemaphoreType.DMA,
                       pltpu.SemaphoreType.DMA,
                       pltpu.SemaphoreType.DMA,
                       pltpu.SemaphoreType.DMA,
                       pltpu.SemaphoreType.DMA,
                       pltpu.VMEM_SHARED((NPAD, width), jnp.float32)],
        compiler_params=_SC_PARAMS,
    )
    def agg(hflat_hbm, src_hbm, dst_hbm, wt_hbm, uout_hbm,
            idx0_v, idx1_v, dst_v, dsts_v, wv, rows_v, prod_v, zero_v,
            s00, s01, s10, s11, sp0, sp1, acc_sh):
        core = lax.axis_index("c")
        sid = lax.axis_index("s")
        sems = ((s00, s01), (s10, s11))
        psems = (sp0, sp1)

        @pl.loop(0, WIN)
        def _(j):
            for q in range(width // 16):
                zero_v[j, pl.ds(q * 16, 16)] = jnp.zeros((16,), jnp.float32)

        @pl.loop(0, npass)
        def _(kk):
            if npass > 1:
                k0 = (npass * core + kk) * nh
                slot = npass * core + kk
                ebase = sid * per_w
            else:
                k0 = 0
                slot = core
                ebase = (core * NS + sid) * per_w

            for t in range(NPAD // NS // WIN):
                pltpu.sync_copy(
                    zero_v, acc_sh.at[pl.ds(sid * (NPAD // NS) + t * WIN,
                                            WIN)])
            plsc.subcore_barrier()

            def prefetch(n, b):
                base = ebase + n * WIN
                pltpu.sync_copy(src_hbm.at[pl.ds(base, WIN)], idx0_v.at[b])
                pltpu.sync_copy(dst_hbm.at[pl.ds(base, WIN)], dst_v.at[b])
                pltpu.sync_copy(wt_hbm.at[k0, pl.ds(base, WIN)],
                                wv.at[b, 0])
                if nh > 1:
                    pltpu.sync_copy(wt_hbm.at[k0 + 1, pl.ds(base, WIN)],
                                    wv.at[b, 1])
                for q in range(WIN // 16):
                    sl = pl.ds(q * 16, 16)
                    if nh > 1:
                        idx1_v[b, sl] = idx0_v[b, sl] + (k0 + 1) * NPAD
                    if npass > 1:
                        idx0_v[b, sl] = idx0_v[b, sl] + k0 * NPAD
                pltpu.make_async_copy(hflat_hbm.at[idx0_v.at[b]],
                                      rows_v.at[b, 0], sems[b][0]).start()
                if nh > 1:
                    pltpu.make_async_copy(hflat_hbm.at[idx1_v.at[b]],
                                          rows_v.at[b, 1], sems[b][1]).start()

            def add_copy(b):
                return pltpu.make_async_copy(
                    prod_v.at[b], acc_sh.at[dsts_v.at[b]], psems[b])

            def consume(b):
                pltpu.make_async_copy(hflat_hbm.at[idx0_v.at[b]],
                                      rows_v.at[b, 0], sems[b][0]).wait()
                if nh > 1:
                    pltpu.make_async_copy(hflat_hbm.at[idx1_v.at[b]],
                                          rows_v.at[b, 1], sems[b][1]).wait()
                add_copy(b).wait()
                for q in range(WIN // 16):
                    sl = pl.ds(q * 16, 16)
                    dsts_v[b, sl] = dst_v[b, sl]

                @pl.loop(0, WIN, step=16)
                def _(j0):
                    for jj in range(16):
                        for i in range(nh):
                            splat = jnp.full((16,), 1.0, jnp.float32) * (
                                wv[b, i, pl.ds(j0, 16)][jj])
                            for q in range(HID // 16):
                                prod_v[b, j0 + jj,
                                       pl.ds(i * HID + q * 16, 16)] = (
                                    rows_v[b, i, j0 + jj, pl.ds(q * 16, 16)]
                                    * splat)

                add_copy(b).start(add=True)

            prefetch(0, 0)
            prefetch(1, 1)
            for bb in range(2):
                for q in range(WIN // 16):
                    sl = pl.ds(q * 16, 16)
                    dsts_v[bb, sl] = dst_v[bb, sl]
            pltpu.make_async_copy(zero_v, acc_sh.at[dsts_v.at[0]],
                                  psems[0]).start(add=True)
            pltpu.make_async_copy(zero_v, acc_sh.at[dsts_v.at[1]],
                                  psems[1]).start(add=True)

            @pl.loop(0, nwin // 2 - 1)
            def _(p):
                n = p * 2
                consume(0)
                prefetch(n + 2, 0)
                consume(1)
                prefetch(n + 3, 1)

            consume(0)
            consume(1)
            add_copy(0).wait()
            add_copy(1).wait()

            plsc.subcore_barrier()
            pltpu.sync_copy(
                acc_sh.at[pl.ds(sid * (NPAD // NS), NPAD // NS)],
                uout_hbm.at[slot, pl.ds(sid * (NPAD // NS), NPAD // NS)])
            plsc.subcore_barrier()

    return agg


def _dense2_body(uout_ref, d0_ref, d1_ref, bg1_ref, wg2_ref, as2_ref,
                 ad2_ref, h2_ref, es2_ref, ed2_ref):
    den16 = d0_ref[0] + d1_ref[0] + 1e-16
    parts = []
    for k in range(HEADS):
        num = uout_ref[k]
        den = den16[:, k:k + 1]
        parts.append(jnp.maximum(num / den + bg1_ref[:, k * HID:(k + 1) * HID],
                                 0.0))
    xg = jnp.concatenate(parts, axis=1)
    h2 = _bf16dot(xg, wg2_ref[...])
    h2_ref[...] = h2
    z = jnp.zeros((BR, 15), jnp.float32)
    es2 = jnp.sum(h2 * as2_ref[...], axis=1, keepdims=True)
    ed2 = jnp.sum(h2 * ad2_ref[...], axis=1, keepdims=True)
    es2_ref[...] = jnp.concatenate([es2, z], axis=1)
    ed2_ref[...] = jnp.concatenate([ed2, z], axis=1)


def _final_body(x1_ref, u0_ref, u1_ref, d0_ref, d1_ref, bg2_ref, wp_ref,
                bp_ref, final_ref, g_ref, gt_ref, sq_ref):
    i = pl.program_id(0)
    num = u0_ref[0] + u1_ref[0]
    den = d0_ref[0][:, 0:1] + d1_ref[0][:, 0:1] + 1e-16
    x2 = num / den + bg2_ref[...]
    fb = jnp.concatenate([x1_ref[...], x2], axis=1)
    final_ref[...] = fb
    g = _bf16dot(fb, wp_ref[...]) + bp_ref[...]
    sq = jnp.sum(g * g, axis=1, keepdims=True)
    row = i * BR + lax.broadcasted_iota(jnp.int32, (BR, 1), 0)
    sq_ref[...] = jnp.where(row < N, sq, 1e30)
    g_ref[...] = g
    gt_ref[...] = g.T


def _knn_body(g_ref, gt_ref, sq_ref, sqrow_ref, final_ref, fblk_ref,
              gln_ref, bln_ref, wo1_ref, wo2_ref, bo_ref,
              out_ref, key_ref, mask_ref):
    dots = _bf16dot(g_ref[...], gt_ref[...])
    d = jnp.maximum(sq_ref[...] + sqrow_ref[...] - 2.0 * dots, 0.0)
    key_ref[...] = lax.bitcast_convert_type(d, jnp.int32)

    lo0 = jnp.zeros((BR, 1), jnp.int32)
    hi0 = jnp.full((BR, 1), 0x7F800000, jnp.int32)

    def bs_step(_, carry):
        lo, hi = carry
        mid = lo + ((hi - lo) >> 1)
        cnt = jnp.sum((key_ref[...] <= mid).astype(jnp.int32), axis=1,
                      keepdims=True)
        pred = cnt >= K
        return jnp.where(pred, lo, mid + 1), jnp.where(pred, mid, hi)

    lo, hi = lax.fori_loop(0, 31, bs_step, (lo0, hi0))

    mask = (key_ref[...] <= hi).astype(jnp.float32)
    cnt = jnp.sum(mask, axis=1, keepdims=True)
    mask_ref[...] = mask

    sim = lax.dot_general(mask_ref[...], final_ref[...],
                          (((1,), (0,)), ((), ())), precision=_HI) / cnt
    mu = jnp.mean(sim, axis=1, keepdims=True)
    var = jnp.mean((sim - mu) ** 2, axis=1, keepdims=True)
    simn = (sim - mu) * lax.rsqrt(var + 1e-5) * gln_ref[...] + bln_ref[...]

    out = (lax.dot_general(fblk_ref[...], wo1_ref[...],
                           (((1,), (0,)), ((), ())), precision=_HI)
           + lax.dot_general(simn, wo2_ref[...],
                             (((1,), (0,)), ((), ())), precision=_HI)
           + bo_ref[...])
    out_ref[...] = out


_gw = _make_gw()
_agg1 = _make_agg(4, EPAD, HID)
_agg2 = _make_agg(1, EPAD // 2, HID)


def _full(x, edge_index, W1, b1, Wg1, as1, ad1, bg1, Wg2, as2, ad2, bg2,
          Wp, bp, g_ln, b_ln, Wo, bo):
    xpad = jnp.zeros((NPAD, F_IN), jnp.float32).at[:N].set(x)
    srcp = jnp.zeros((EPAD,), jnp.int32).at[:E].set(edge_index[0])
    dstp = jnp.zeros((EPAD,), jnp.int32).at[:E].set(edge_index[1])

    x1, h, es16, ed16 = pl.pallas_call(
        _dense1_body,
        grid=(NBLK,),
        in_specs=[
            pl.BlockSpec((BR, F_IN), lambda i: (i, 0)),
            pl.BlockSpec((F_IN, HID), lambda i: (0, 0)),
            pl.BlockSpec((1, HID), lambda i: (0, 0)),
            pl.BlockSpec((F_IN, HEADS * HID), lambda i: (0, 0)),
            pl.BlockSpec((HEADS, HID), lambda i: (0, 0)),
            pl.BlockSpec((HEADS, HID), lambda i: (0, 0)),
        ],
        out_specs=[
            pl.BlockSpec((BR, HID), lambda i: (i, 0)),
            pl.BlockSpec((HEADS, BR, HID), lambda i: (0, i, 0)),
            pl.BlockSpec((BR, 16), lambda i: (i, 0)),
            pl.BlockSpec((BR, 16), lambda i: (i, 0)),
        ],
        out_shape=[
            jax.ShapeDtypeStruct((NPAD, HID), jnp.float32),
            jax.ShapeDtypeStruct((HEADS, NPAD, HID), jnp.float32),
            jax.ShapeDtypeStruct((NPAD, 16), jnp.float32),
            jax.ShapeDtypeStruct((NPAD, 16), jnp.float32),
        ],
    )(xpad, W1, b1.reshape(1, HID), Wg1, as1, ad1)

    w16_1, den1 = _gw(es16, ed16, srcp, dstp)

    wt1 = pl.pallas_call(
        _w1_body,
        grid=(NEB,),
        in_specs=[pl.BlockSpec((EBLK, 16), lambda i: (i, 0))],
        out_specs=pl.BlockSpec((HEADS, EBLK), lambda i: (0, i)),
        out_shape=jax.ShapeDtypeStruct((HEADS, EPAD), jnp.float32),
    )(w16_1)

    hflat = h.reshape(HEADS * NPAD, HID)
    uout1 = _agg1(hflat, srcp, dstp, wt1)

    h2, es2, ed2 = pl.pallas_call(
        _dense2_body,
        grid=(NBLK,),
        in_specs=[
            pl.BlockSpec((HEADS, BR, HID), lambda i: (0, i, 0)),
            pl.BlockSpec((1, BR, 16), lambda i: (0, i, 0)),
            pl.BlockSpec((1, BR, 16), lambda i: (1, i, 0)),
            pl.BlockSpec((1, HEADS * HID), lambda i: (0, 0)),
            pl.BlockSpec((HEADS * HID, HID), lambda i: (0, 0)),
            pl.BlockSpec((1, HID), lambda i: (0, 0)),
            pl.BlockSpec((1, HID), lambda i: (0, 0)),
        ],
        out_specs=[pl.BlockSpec((BR, HID), lambda i: (i, 0)),
                   pl.BlockSpec((BR, 16), lambda i: (i, 0)),
                   pl.BlockSpec((BR, 16), lambda i: (i, 0))],
        out_shape=[jax.ShapeDtypeStruct((NPAD, HID), jnp.float32),
                   jax.ShapeDtypeStruct((NPAD, 16), jnp.float32),
                   jax.ShapeDtypeStruct((NPAD, 16), jnp.float32)],
    )(uout1, den1, den1, bg1.reshape(1, -1), Wg2, as2, ad2)

    w16_2, den2 = _gw(es2, ed2, srcp, dstp)

    wt2 = pl.pallas_call(
        _w2_body,
        grid=(NEB,),
        in_specs=[pl.BlockSpec((EBLK, 16), lambda i: (i, 0))],
        out_specs=pl.BlockSpec((1, EBLK), lambda i: (0, i)),
        out_shape=jax.ShapeDtypeStruct((1, EPAD), jnp.float32),
    )(w16_2)

    uout2 = _agg2(h2, srcp, dstp, wt2)

    final, g, gt, sq = pl.pallas_call(
        _final_body,
        grid=(NBLK,),
        in_specs=[
            pl.BlockSpec((BR, HID), lambda i: (i, 0)),
            pl.BlockSpec((1, BR, HID), lambda i: (0, i, 0)),
            pl.BlockSpec((1, BR, HID), lambda i: (1, i, 0)),
            pl.BlockSpec((1, BR, 16), lambda i: (0, i, 0)),
            pl.BlockSpec((1, BR, 16), lambda i: (1, i, 0)),
            pl.BlockSpec((1, HID), lambda i: (0, 0)),
            pl.BlockSpec((2 * HID, PROJ), lambda i: (0, 0)),
            pl.BlockSpec((1, PROJ), lambda i: (0, 0)),
        ],
        out_specs=[
            pl.BlockSpec((BR, 2 * HID), lambda i: (i, 0)),
            pl.BlockSpec((BR, PROJ), lambda i: (i, 0)),
            pl.BlockSpec((PROJ, BR), lambda i: (0, i)),
            pl.BlockSpec((BR, 1), lambda i: (i, 0)),
        ],
        out_shape=[
            jax.ShapeDtypeStruct((NPAD, 2 * HID), jnp.float32),
            jax.ShapeDtypeStruct((NPAD, PROJ), jnp.float32),
            jax.ShapeDtypeStruct((PROJ, NPAD), jnp.float32),
            jax.ShapeDtypeStruct((NPAD, 1), jnp.float32),
        ],
    )(x1, uout2, uout2, den2, den2, bg2.reshape(1, HID), Wp,
      bp.reshape(1, PROJ))

    sqrow = sq.reshape(1, NPAD)

    out = pl.pallas_call(
        _knn_body,
        grid=(NBLK,),
        in_specs=[
            pl.BlockSpec((BR, PROJ), lambda i: (i, 0)),
            pl.BlockSpec((PROJ, NPAD), lambda i: (0, 0)),
            pl.BlockSpec((BR, 1), lambda i: (i, 0)),
            pl.BlockSpec((1, NPAD), lambda i: (0, 0)),
            pl.BlockSpec((NPAD, 2 * HID), lambda i: (0, 0)),
            pl.BlockSpec((BR, 2 * HID), lambda i: (i, 0)),
            pl.BlockSpec((1, 2 * HID), lambda i: (0, 0)),
            pl.BlockSpec((1, 2 * HID), lambda i: (0, 0)),
            pl.BlockSpec((2 * HID, HID), lambda i: (0, 0)),
            pl.BlockSpec((2 * HID, HID), lambda i: (0, 0)),
            pl.BlockSpec((1, HID), lambda i: (0, 0)),
        ],
        out_specs=pl.BlockSpec((BR, HID), lambda i: (i, 0)),
        out_shape=jax.ShapeDtypeStruct((NPAD, HID), jnp.float32),
        scratch_shapes=[
            pltpu.VMEM((BR, NPAD), jnp.int32),
            pltpu.VMEM((BR, NPAD), jnp.float32),
        ],
    )(g, gt, sq, sqrow, final, final, g_ln.reshape(1, -1),
      b_ln.reshape(1, -1), Wo[:2 * HID], Wo[2 * HID:], bo.reshape(1, HID))

    return out[:N]


def kernel(x, edge_index, W1, b1, Wg1, as1, ad1, bg1, Wg2, as2, ad2, bg2,
           Wp, bp, g_ln, b_ln, Wo, bo):
    return _full(x, edge_index, W1, b1, Wg1, as1, ad1, bg1, Wg2, as2, ad2,
                 bg2, Wp, bp, g_ln, b_ln, Wo, bo)


# speedup vs baseline: 1.1298x; 1.0135x over previous
import functools

import jax
import jax.numpy as jnp
from jax import lax
from jax.experimental import pallas as pl
from jax.experimental.pallas import tpu as pltpu
from jax.experimental.pallas import tpu_sc as plsc

N = 10000
F_IN = 128
HID = 64
HEADS = 8
PROJ = 32
E = 160000
K = 50

NPAD = 10240
BR = 256
NBLK = NPAD // BR

EPAD = 163840
EBLK = 8192
NEB = EPAD // EBLK

NC, NS = 2, 16
NW = NC * NS
WIN = 128

_SC_PARAMS = pltpu.CompilerParams(use_tc_tiling_on_sc=False)

_HI = jax.lax.Precision.HIGHEST


def _bf16dot(a, b):
    return lax.dot_general(a.astype(jnp.bfloat16), b.astype(jnp.bfloat16),
                           (((1,), (0,)), ((), ())),
                           preferred_element_type=jnp.float32)


def _dense1_body(x_ref, w1_ref, b1_ref, wg1_ref, as1_ref, ad1_ref,
                 x1_ref, h_ref, es_ref, ed_ref):
    xb = x_ref[...]
    x1_ref[...] = jnp.maximum(_bf16dot(xb, w1_ref[...]) + b1_ref[...], 0.0)
    z = jnp.zeros((BR, 8), jnp.float32)
    es_ref[:, 8:16] = z
    ed_ref[:, 8:16] = z
    for k in range(HEADS):
        hk = _bf16dot(xb, wg1_ref[:, k * HID:(k + 1) * HID])
        h_ref[k] = hk
        es_ref[:, k:k + 1] = jnp.sum(hk * as1_ref[k:k + 1, :], axis=1,
                                     keepdims=True)
        ed_ref[:, k:k + 1] = jnp.sum(hk * ad1_ref[k:k + 1, :], axis=1,
                                     keepdims=True)


def _make_gw():
    per_w = EPAD // NW
    nwin = per_w // WIN
    mesh = plsc.VectorSubcoreMesh(core_axis_name="c", subcore_axis_name="s")
    assert nwin % 2 == 0 and nwin >= 4

    @functools.partial(
        pl.kernel, mesh=mesh,
        out_type=[jax.ShapeDtypeStruct((EPAD, 16), jnp.float32),
                  jax.ShapeDtypeStruct((NC, NPAD, 16), jnp.float32)],
        scratch_types=[pltpu.VMEM((2, WIN), jnp.int32),
                       pltpu.VMEM((2, WIN), jnp.int32),
                       pltpu.VMEM((2, WIN, 16), jnp.float32),
                       pltpu.VMEM((2, WIN, 16), jnp.float32),
                       pltpu.VMEM((WIN, 16), jnp.float32),
                       pltpu.VMEM((WIN, 16), jnp.float32),
                       pltpu.SemaphoreType.DMA,
                       pltpu.SemaphoreType.DMA,
                       pltpu.SemaphoreType.DMA,
                       pltpu.SemaphoreType.DMA,
                       pltpu.VMEM_SHARED((NPAD, 16), jnp.float32)],
        compiler_params=_SC_PARAMS,
    )
    def gw(es_hbm, ed_hbm, src_hbm, dst_hbm, w_hbm, den_hbm,
           idxs_v, idxd_v, ra_v, rb_v, w_v, zero_v, sa0, sa1, sb0, sb1,
           den_sh):
        core = lax.axis_index("c")
        sid = lax.axis_index("s")
        ebase = (core * NS + sid) * per_w
        sems = ((sa0, sb0), (sa1, sb1))

        @pl.loop(0, WIN)
        def _(j):
            zero_v[j, pl.ds(0, 16)] = jnp.zeros((16,), jnp.float32)

        for t in range(NPAD // NS // WIN):
            pltpu.sync_copy(
                zero_v, den_sh.at[pl.ds(sid * (NPAD // NS) + t * WIN, WIN)])
        plsc.subcore_barrier()

        def prefetch(n, b):
            base = ebase + n * WIN
            pltpu.sync_copy(src_hbm.at[pl.ds(base, WIN)], idxs_v.at[b])
            pltpu.sync_copy(dst_hbm.at[pl.ds(base, WIN)], idxd_v.at[b])
            pltpu.make_async_copy(es_hbm.at[idxs_v.at[b]], ra_v.at[b],
                                  sems[b][0]).start()
            pltpu.make_async_copy(ed_hbm.at[idxd_v.at[b]], rb_v.at[b],
                                  sems[b][1]).start()

        def consume(n, b):
            base = ebase + n * WIN
            pltpu.make_async_copy(es_hbm.at[idxs_v.at[b]], ra_v.at[b],
                                  sems[b][0]).wait()
            pltpu.make_async_copy(ed_hbm.at[idxd_v.at[b]], rb_v.at[b],
                                  sems[b][1]).wait()

            @pl.loop(0, WIN)
            def _(j):
                e = ra_v[b, j, pl.ds(0, 16)] + rb_v[b, j, pl.ds(0, 16)]
                e = jnp.where(e > 0, e, 0.2 * e)
                w = jnp.exp(e)
                w = jnp.where(base + j < E, w, jnp.zeros((16,), jnp.float32))
                w_v[j, pl.ds(0, 16)] = w

            pltpu.sync_copy(w_v, w_hbm.at[pl.ds(base, WIN)])
            pltpu.sync_copy(w_v, den_sh.at[idxd_v.at[b]], add=True)

        prefetch(0, 0)
        prefetch(1, 1)

        @pl.loop(0, nwin // 2 - 1)
        def _(p):
            n = p * 2
            consume(n, 0)
            prefetch(n + 2, 0)
            consume(n + 1, 1)
            prefetch(n + 3, 1)

        consume(nwin - 2, 0)
        consume(nwin - 1, 1)

        plsc.subcore_barrier()
        pltpu.sync_copy(
            den_sh.at[pl.ds(sid * (NPAD // NS), NPAD // NS)],
            den_hbm.at[core, pl.ds(sid * (NPAD // NS), NPAD // NS)])
        plsc.subcore_barrier()

    return gw


def _w1_body(w16_ref, wt_ref):
    wt_ref[...] = w16_ref[:, 0:HEADS].T


def _w2_body(w16_ref, wt_ref):
    wt_ref[...] = w16_ref[:, 0:1].T


def _make_agg(pairs_per_core, edges_per_pass, width):
    per_w = edges_per_pass // NS
    nwin = per_w // WIN
    npass = pairs_per_core
    nslot = npass * NC
    nh = width // HID
    mesh = plsc.VectorSubcoreMesh(core_axis_name="c", subcore_axis_name="s")
    assert nwin % 2 == 0 and nwin >= 4

    @functools.partial(
        pl.kernel, mesh=mesh,
        out_type=jax.ShapeDtypeStruct((nslot, NPAD, width), jnp.float32),
        scratch_types=[pltpu.VMEM((2, WIN), jnp.int32),
                       pltpu.VMEM((2, WIN), jnp.int32),
                       pltpu.VMEM((2, WIN), jnp.int32),
                       pltpu.VMEM((2, WIN), jnp.int32),
                       pltpu.VMEM((2, 2, WIN), jnp.float32),
                       pltpu.VMEM((2, 2, WIN, HID), jnp.float32),
                       pltpu.VMEM((2, WIN, width), jnp.float32),
                       pltpu.VMEM((WIN, width), jnp.float32),
                       pltpu.SemaphoreType.DMA,
                       pltpu.SemaphoreType.DMA,
                       pltpu.SemaphoreType.DMA,
                       pltpu.SemaphoreType.DMA,
                       pltpu.SemaphoreType.DMA,
                       pltpu.SemaphoreType.DMA,
                       pltpu.VMEM_SHARED((NPAD, width), jnp.float32)],
        compiler_params=_SC_PARAMS,
    )
    def agg(hflat_hbm, src_hbm, dst_hbm, wt_hbm, uout_hbm,
            idx0_v, idx1_v, dst_v, dsts_v, wv, rows_v, prod_v, zero_v,
            s00, s01, s10, s11, sp0, sp1, acc_sh):
        core = lax.axis_index("c")
        sid = lax.axis_index("s")
        sems = ((s00, s01), (s10, s11))
        psems = (sp0, sp1)

        @pl.loop(0, WIN)
        def _(j):
            for q in range(width // 16):
                zero_v[j, pl.ds(q * 16, 16)] = jnp.zeros((16,), jnp.float32)

        @pl.loop(0, npass)
        def _(kk):
            if npass > 1:
                k0 = (npass * core + kk) * nh
                slot = npass * core + kk
                ebase = sid * per_w
            else:
                k0 = 0
                slot = core
                ebase = (core * NS + sid) * per_w

            for t in range(NPAD // NS // WIN):
                pltpu.sync_copy(
                    zero_v, acc_sh.at[pl.ds(sid * (NPAD // NS) + t * WIN,
                                            WIN)])
            plsc.subcore_barrier()

            def prefetch(n, b):
                base = ebase + n * WIN
                pltpu.sync_copy(src_hbm.at[pl.ds(base, WIN)], idx0_v.at[b])
                pltpu.sync_copy(dst_hbm.at[pl.ds(base, WIN)], dst_v.at[b])
                pltpu.sync_copy(wt_hbm.at[k0, pl.ds(base, WIN)],
                                wv.at[b, 0])
                if nh > 1:
                    pltpu.sync_copy(wt_hbm.at[k0 + 1, pl.ds(base, WIN)],
                                    wv.at[b, 1])
                for q in range(WIN // 16):
                    sl = pl.ds(q * 16, 16)
                    if nh > 1:
                        idx1_v[b, sl] = idx0_v[b, sl] + (k0 + 1) * NPAD
                    if npass > 1:
                        idx0_v[b, sl] = idx0_v[b, sl] + k0 * NPAD
                pltpu.make_async_copy(hflat_hbm.at[idx0_v.at[b]],
                                      rows_v.at[b, 0], sems[b][0]).start()
                if nh > 1:
                    pltpu.make_async_copy(hflat_hbm.at[idx1_v.at[b]],
                                          rows_v.at[b, 1], sems[b][1]).start()

            def add_copy(b):
                return pltpu.make_async_copy(
                    prod_v.at[b], acc_sh.at[dsts_v.at[b]], psems[b])

            def consume(b):
                pltpu.make_async_copy(hflat_hbm.at[idx0_v.at[b]],
                                      rows_v.at[b, 0], sems[b][0]).wait()
                if nh > 1:
                    pltpu.make_async_copy(hflat_hbm.at[idx1_v.at[b]],
                                          rows_v.at[b, 1], sems[b][1]).wait()
                add_copy(b).wait()
                for q in range(WIN // 16):
                    sl = pl.ds(q * 16, 16)
                    dsts_v[b, sl] = dst_v[b, sl]

                @pl.loop(0, WIN, step=16)
                def _(j0):
                    for jj in range(16):
                        for i in range(nh):
                            splat = jnp.full((16,), 1.0, jnp.float32) * (
                                wv[b, i, pl.ds(j0, 16)][jj])
                            for q in range(HID // 16):
                                prod_v[b, j0 + jj,
                                       pl.ds(i * HID + q * 16, 16)] = (
                                    rows_v[b, i, j0 + jj, pl.ds(q * 16, 16)]
                                    * splat)

                add_copy(b).start(add=True)

            prefetch(0, 0)
            prefetch(1, 1)
            for bb in range(2):
                for q in range(WIN // 16):
                    sl = pl.ds(q * 16, 16)
                    dsts_v[bb, sl] = dst_v[bb, sl]
            pltpu.make_async_copy(zero_v, acc_sh.at[dsts_v.at[0]],
                                  psems[0]).start(add=True)
            pltpu.make_async_copy(zero_v, acc_sh.at[dsts_v.at[1]],
                                  psems[1]).start(add=True)

            @pl.loop(0, nwin // 2 - 1)
            def _(p):
                n = p * 2
                consume(0)
                prefetch(n + 2, 0)
                consume(1)
                prefetch(n + 3, 1)

            consume(0)
            consume(1)
            add_copy(0).wait()
            add_copy(1).wait()

            plsc.subcore_barrier()
            pltpu.sync_copy(
                acc_sh.at[pl.ds(sid * (NPAD // NS), NPAD // NS)],
                uout_hbm.at[slot, pl.ds(sid * (NPAD // NS), NPAD // NS)])
            plsc.subcore_barrier()

    return agg


def _dense2_body(uout_ref, d0_ref, d1_ref, bg1_ref, wg2_ref, as2_ref,
                 ad2_ref, h2_ref, es2_ref, ed2_ref):
    den16 = d0_ref[0] + d1_ref[0] + 1e-16
    parts = []
    for k in range(HEADS):
        num = uout_ref[k]
        den = den16[:, k:k + 1]
        parts.append(jnp.maximum(num / den + bg1_ref[:, k * HID:(k + 1) * HID],
                                 0.0))
    xg = jnp.concatenate(parts, axis=1)
    h2 = _bf16dot(xg, wg2_ref[...])
    h2_ref[...] = h2
    z = jnp.zeros((BR, 15), jnp.float32)
    es2 = jnp.sum(h2 * as2_ref[...], axis=1, keepdims=True)
    ed2 = jnp.sum(h2 * ad2_ref[...], axis=1, keepdims=True)
    es2_ref[...] = jnp.concatenate([es2, z], axis=1)
    ed2_ref[...] = jnp.concatenate([ed2, z], axis=1)


def _final_body(x1_ref, u0_ref, u1_ref, d0_ref, d1_ref, bg2_ref, wp_ref,
                bp_ref, final_ref, g_ref, gt_ref, sq_ref):
    i = pl.program_id(0)
    num = u0_ref[0] + u1_ref[0]
    den = d0_ref[0][:, 0:1] + d1_ref[0][:, 0:1] + 1e-16
    x2 = num / den + bg2_ref[...]
    fb = jnp.concatenate([x1_ref[...], x2], axis=1)
    final_ref[...] = fb
    g = _bf16dot(fb, wp_ref[...]) + bp_ref[...]
    sq = jnp.sum(g * g, axis=1, keepdims=True)
    row = i * BR + lax.broadcasted_iota(jnp.int32, (BR, 1), 0)
    sq_ref[...] = jnp.where(row < N, sq, 1e30)
    g_ref[...] = g
    gt_ref[...] = g.T


BK = 512
NBK = NPAD // BK


def _knn_body(g_ref, gt_ref, sq_ref, sqrow_ref, final_ref, fblk_ref,
              gln_ref, bln_ref, wo1_ref, wo2_ref, bo_ref,
              out_ref, key_ref, mask_ref):
    dots = _bf16dot(g_ref[...], gt_ref[...])
    d = jnp.maximum(sq_ref[...] + sqrow_ref[...] - 2.0 * dots, 0.0)
    key_ref[...] = lax.bitcast_convert_type(d, jnp.int32)

    lo0 = jnp.zeros((BK, 1), jnp.int32)
    hi0 = jnp.full((BK, 1), 0x7F800000, jnp.int32)

    def bs_step(_, carry):
        lo, hi = carry
        mid = lo + ((hi - lo) >> 1)
        cnt = jnp.sum((key_ref[...] <= mid).astype(jnp.int32), axis=1,
                      keepdims=True)
        pred = cnt >= K
        return jnp.where(pred, lo, mid + 1), jnp.where(pred, mid, hi)

    lo, hi = lax.fori_loop(0, 31, bs_step, (lo0, hi0))

    mask = (key_ref[...] <= hi).astype(jnp.float32)
    cnt = jnp.sum(mask, axis=1, keepdims=True)
    mask_ref[...] = mask

    sim = lax.dot_general(mask_ref[...], final_ref[...],
                          (((1,), (0,)), ((), ())), precision=_HI) / cnt
    mu = jnp.mean(sim, axis=1, keepdims=True)
    var = jnp.mean((sim - mu) ** 2, axis=1, keepdims=True)
    simn = (sim - mu) * lax.rsqrt(var + 1e-5) * gln_ref[...] + bln_ref[...]

    out = (lax.dot_general(fblk_ref[...], wo1_ref[...],
                           (((1,), (0,)), ((), ())), precision=_HI)
           + lax.dot_general(simn, wo2_ref[...],
                             (((1,), (0,)), ((), ())), precision=_HI)
           + bo_ref[...])
    out_ref[...] = out


_gw = _make_gw()
_agg1 = _make_agg(4, EPAD, HID)
_agg2 = _make_agg(1, EPAD // 2, HID)


def _full(x, edge_index, W1, b1, Wg1, as1, ad1, bg1, Wg2, as2, ad2, bg2,
          Wp, bp, g_ln, b_ln, Wo, bo):
    xpad = jnp.zeros((NPAD, F_IN), jnp.float32).at[:N].set(x)
    srcp = jnp.zeros((EPAD,), jnp.int32).at[:E].set(edge_index[0])
    dstp = jnp.zeros((EPAD,), jnp.int32).at[:E].set(edge_index[1])

    x1, h, es16, ed16 = pl.pallas_call(
        _dense1_body,
        grid=(NBLK,),
        in_specs=[
            pl.BlockSpec((BR, F_IN), lambda i: (i, 0)),
            pl.BlockSpec((F_IN, HID), lambda i: (0, 0)),
            pl.BlockSpec((1, HID), lambda i: (0, 0)),
            pl.BlockSpec((F_IN, HEADS * HID), lambda i: (0, 0)),
            pl.BlockSpec((HEADS, HID), lambda i: (0, 0)),
            pl.BlockSpec((HEADS, HID), lambda i: (0, 0)),
        ],
        out_specs=[
            pl.BlockSpec((BR, HID), lambda i: (i, 0)),
            pl.BlockSpec((HEADS, BR, HID), lambda i: (0, i, 0)),
            pl.BlockSpec((BR, 16), lambda i: (i, 0)),
            pl.BlockSpec((BR, 16), lambda i: (i, 0)),
        ],
        out_shape=[
            jax.ShapeDtypeStruct((NPAD, HID), jnp.float32),
            jax.ShapeDtypeStruct((HEADS, NPAD, HID), jnp.float32),
            jax.ShapeDtypeStruct((NPAD, 16), jnp.float32),
            jax.ShapeDtypeStruct((NPAD, 16), jnp.float32),
        ],
    )(xpad, W1, b1.reshape(1, HID), Wg1, as1, ad1)

    w16_1, den1 = _gw(es16, ed16, srcp, dstp)

    wt1 = pl.pallas_call(
        _w1_body,
        grid=(NEB,),
        in_specs=[pl.BlockSpec((EBLK, 16), lambda i: (i, 0))],
        out_specs=pl.BlockSpec((HEADS, EBLK), lambda i: (0, i)),
        out_shape=jax.ShapeDtypeStruct((HEADS, EPAD), jnp.float32),
    )(w16_1)

    hflat = h.reshape(HEADS * NPAD, HID)
    uout1 = _agg1(hflat, srcp, dstp, wt1)

    h2, es2, ed2 = pl.pallas_call(
        _dense2_body,
        grid=(NBLK,),
        in_specs=[
            pl.BlockSpec((HEADS, BR, HID), lambda i: (0, i, 0)),
            pl.BlockSpec((1, BR, 16), lambda i: (0, i, 0)),
            pl.BlockSpec((1, BR, 16), lambda i: (1, i, 0)),
            pl.BlockSpec((1, HEADS * HID), lambda i: (0, 0)),
            pl.BlockSpec((HEADS * HID, HID), lambda i: (0, 0)),
            pl.BlockSpec((1, HID), lambda i: (0, 0)),
            pl.BlockSpec((1, HID), lambda i: (0, 0)),
        ],
        out_specs=[pl.BlockSpec((BR, HID), lambda i: (i, 0)),
                   pl.BlockSpec((BR, 16), lambda i: (i, 0)),
                   pl.BlockSpec((BR, 16), lambda i: (i, 0))],
        out_shape=[jax.ShapeDtypeStruct((NPAD, HID), jnp.float32),
                   jax.ShapeDtypeStruct((NPAD, 16), jnp.float32),
                   jax.ShapeDtypeStruct((NPAD, 16), jnp.float32)],
    )(uout1, den1, den1, bg1.reshape(1, -1), Wg2, as2, ad2)

    w16_2, den2 = _gw(es2, ed2, srcp, dstp)

    wt2 = pl.pallas_call(
        _w2_body,
        grid=(NEB,),
        in_specs=[pl.BlockSpec((EBLK, 16), lambda i: (i, 0))],
        out_specs=pl.BlockSpec((1, EBLK), lambda i: (0, i)),
        out_shape=jax.ShapeDtypeStruct((1, EPAD), jnp.float32),
    )(w16_2)

    uout2 = _agg2(h2, srcp, dstp, wt2)

    final, g, gt, sq = pl.pallas_call(
        _final_body,
        grid=(NBLK,),
        in_specs=[
            pl.BlockSpec((BR, HID), lambda i: (i, 0)),
            pl.BlockSpec((1, BR, HID), lambda i: (0, i, 0)),
            pl.BlockSpec((1, BR, HID), lambda i: (1, i, 0)),
            pl.BlockSpec((1, BR, 16), lambda i: (0, i, 0)),
            pl.BlockSpec((1, BR, 16), lambda i: (1, i, 0)),
            pl.BlockSpec((1, HID), lambda i: (0, 0)),
            pl.BlockSpec((2 * HID, PROJ), lambda i: (0, 0)),
            pl.BlockSpec((1, PROJ), lambda i: (0, 0)),
        ],
        out_specs=[
            pl.BlockSpec((BR, 2 * HID), lambda i: (i, 0)),
            pl.BlockSpec((BR, PROJ), lambda i: (i, 0)),
            pl.BlockSpec((PROJ, BR), lambda i: (0, i)),
            pl.BlockSpec((BR, 1), lambda i: (i, 0)),
        ],
        out_shape=[
            jax.ShapeDtypeStruct((NPAD, 2 * HID), jnp.float32),
            jax.ShapeDtypeStruct((NPAD, PROJ), jnp.float32),
            jax.ShapeDtypeStruct((PROJ, NPAD), jnp.float32),
            jax.ShapeDtypeStruct((NPAD, 1), jnp.float32),
        ],
    )(x1, uout2, uout2, den2, den2, bg2.reshape(1, HID), Wp,
      bp.reshape(1, PROJ))

    sqrow = sq.reshape(1, NPAD)

    out = pl.pallas_call(
        _knn_body,
        grid=(NBK,),
        in_specs=[
            pl.BlockSpec((BK, PROJ), lambda i: (i, 0)),
            pl.BlockSpec((PROJ, NPAD), lambda i: (0, 0)),
            pl.BlockSpec((BK, 1), lambda i: (i, 0)),
            pl.BlockSpec((1, NPAD), lambda i: (0, 0)),
            pl.BlockSpec((NPAD, 2 * HID), lambda i: (0, 0)),
            pl.BlockSpec((BK, 2 * HID), lambda i: (i, 0)),
            pl.BlockSpec((1, 2 * HID), lambda i: (0, 0)),
            pl.BlockSpec((1, 2 * HID), lambda i: (0, 0)),
            pl.BlockSpec((2 * HID, HID), lambda i: (0, 0)),
            pl.BlockSpec((2 * HID, HID), lambda i: (0, 0)),
            pl.BlockSpec((1, HID), lambda i: (0, 0)),
        ],
        out_specs=pl.BlockSpec((BK, HID), lambda i: (i, 0)),
        out_shape=jax.ShapeDtypeStruct((NPAD, HID), jnp.float32),
        scratch_shapes=[
            pltpu.VMEM((BK, NPAD), jnp.int32),
            pltpu.VMEM((BK, NPAD), jnp.float32),
        ],
    )(g, gt, sq, sqrow, final, final, g_ln.reshape(1, -1),
      b_ln.reshape(1, -1), Wo[:2 * HID], Wo[2 * HID:], bo.reshape(1, HID))

    return out[:N]


def kernel(x, edge_index, W1, b1, Wg1, as1, ad1, bg1, Wg2, as2, ad2, bg2,
           Wp, bp, g_ln, b_ln, Wo, bo):
    return _full(x, edge_index, W1, b1, Wg1, as1, ad1, bg1, Wg2, as2, ad2,
                 bg2, Wp, bp, g_ln, b_ln, Wo, bo)


# speedup vs baseline: 1.1299x; 1.0000x over previous
import functools

import jax
import jax.numpy as jnp
from jax import lax
from jax.experimental import pallas as pl
from jax.experimental.pallas import tpu as pltpu
from jax.experimental.pallas import tpu_sc as plsc

N = 10000
F_IN = 128
HID = 64
HEADS = 8
PROJ = 32
E = 160000
K = 50

NPAD = 10240
BR = 256
NBLK = NPAD // BR

EPAD = 163840
EBLK = 8192
NEB = EPAD // EBLK

NC, NS = 2, 16
NW = NC * NS
WIN = 128

_SC_PARAMS = pltpu.CompilerParams(use_tc_tiling_on_sc=False)

_HI = jax.lax.Precision.HIGHEST


def _bf16dot(a, b):
    return lax.dot_general(a.astype(jnp.bfloat16), b.astype(jnp.bfloat16),
                           (((1,), (0,)), ((), ())),
                           preferred_element_type=jnp.float32)


def _dense1_body(x_ref, w1_ref, b1_ref, wg1_ref, as1_ref, ad1_ref,
                 x1_ref, h_ref, es_ref, ed_ref):
    xb = x_ref[...]
    x1_ref[...] = jnp.maximum(_bf16dot(xb, w1_ref[...]) + b1_ref[...], 0.0)
    z = jnp.zeros((BR, 8), jnp.float32)
    es_ref[:, 8:16] = z
    ed_ref[:, 8:16] = z
    for k in range(HEADS):
        hk = _bf16dot(xb, wg1_ref[:, k * HID:(k + 1) * HID])
        h_ref[k] = hk
        es_ref[:, k:k + 1] = jnp.sum(hk * as1_ref[k:k + 1, :], axis=1,
                                     keepdims=True)
        ed_ref[:, k:k + 1] = jnp.sum(hk * ad1_ref[k:k + 1, :], axis=1,
                                     keepdims=True)


def _make_gw():
    per_w = EPAD // NW
    nwin = per_w // WIN
    mesh = plsc.VectorSubcoreMesh(core_axis_name="c", subcore_axis_name="s")
    assert nwin % 2 == 0 and nwin >= 4

    @functools.partial(
        pl.kernel, mesh=mesh,
        out_type=[jax.ShapeDtypeStruct((EPAD, 16), jnp.float32),
                  jax.ShapeDtypeStruct((NC, NPAD, 16), jnp.float32)],
        scratch_types=[pltpu.VMEM((2, WIN), jnp.int32),
                       pltpu.VMEM((2, WIN), jnp.int32),
                       pltpu.VMEM((2, WIN, 16), jnp.float32),
                       pltpu.VMEM((2, WIN, 16), jnp.float32),
                       pltpu.VMEM((WIN, 16), jnp.float32),
                       pltpu.VMEM((WIN, 16), jnp.float32),
                       pltpu.SemaphoreType.DMA,
                       pltpu.SemaphoreType.DMA,
                       pltpu.SemaphoreType.DMA,
                       pltpu.SemaphoreType.DMA,
                       pltpu.VMEM_SHARED((NPAD, 16), jnp.float32)],
        compiler_params=_SC_PARAMS,
    )
    def gw(es_hbm, ed_hbm, src_hbm, dst_hbm, w_hbm, den_hbm,
           idxs_v, idxd_v, ra_v, rb_v, w_v, zero_v, sa0, sa1, sb0, sb1,
           den_sh):
        core = lax.axis_index("c")
        sid = lax.axis_index("s")
        ebase = (core * NS + sid) * per_w
        sems = ((sa0, sb0), (sa1, sb1))

        @pl.loop(0, WIN)
        def _(j):
            zero_v[j, pl.ds(0, 16)] = jnp.zeros((16,), jnp.float32)

        for t in range(NPAD // NS // WIN):
            pltpu.sync_copy(
                zero_v, den_sh.at[pl.ds(sid * (NPAD // NS) + t * WIN, WIN)])
        plsc.subcore_barrier()

        def prefetch(n, b):
            base = ebase + n * WIN
            pltpu.sync_copy(src_hbm.at[pl.ds(base, WIN)], idxs_v.at[b])
            pltpu.sync_copy(dst_hbm.at[pl.ds(base, WIN)], idxd_v.at[b])
            pltpu.make_async_copy(es_hbm.at[idxs_v.at[b]], ra_v.at[b],
                                  sems[b][0]).start()
            pltpu.make_async_copy(ed_hbm.at[idxd_v.at[b]], rb_v.at[b],
                                  sems[b][1]).start()

        def consume(n, b):
            base = ebase + n * WIN
            pltpu.make_async_copy(es_hbm.at[idxs_v.at[b]], ra_v.at[b],
                                  sems[b][0]).wait()
            pltpu.make_async_copy(ed_hbm.at[idxd_v.at[b]], rb_v.at[b],
                                  sems[b][1]).wait()

            @pl.loop(0, WIN)
            def _(j):
                e = ra_v[b, j, pl.ds(0, 16)] + rb_v[b, j, pl.ds(0, 16)]
                e = jnp.where(e > 0, e, 0.2 * e)
                w = jnp.exp(e)
                w = jnp.where(base + j < E, w, jnp.zeros((16,), jnp.float32))
                w_v[j, pl.ds(0, 16)] = w

            pltpu.sync_copy(w_v, w_hbm.at[pl.ds(base, WIN)])
            pltpu.sync_copy(w_v, den_sh.at[idxd_v.at[b]], add=True)

        prefetch(0, 0)
        prefetch(1, 1)

        @pl.loop(0, nwin // 2 - 1)
        def _(p):
            n = p * 2
            consume(n, 0)
            prefetch(n + 2, 0)
            consume(n + 1, 1)
            prefetch(n + 3, 1)

        consume(nwin - 2, 0)
        consume(nwin - 1, 1)

        plsc.subcore_barrier()
        pltpu.sync_copy(
            den_sh.at[pl.ds(sid * (NPAD // NS), NPAD // NS)],
            den_hbm.at[core, pl.ds(sid * (NPAD // NS), NPAD // NS)])
        plsc.subcore_barrier()

    return gw


def _w1_body(w16_ref, wt_ref):
    wt_ref[...] = w16_ref[:, 0:HEADS].T


def _w2_body(w16_ref, wt_ref):
    wt_ref[...] = w16_ref[:, 0:1].T


def _make_agg(passes_per_core, edges_per_pass, width):
    per_w = edges_per_pass // NS
    nwin = per_w // WIN
    npass = passes_per_core
    nslot = npass * NC
    nh = width // HID
    mesh = plsc.VectorSubcoreMesh(core_axis_name="c", subcore_axis_name="s")
    assert nwin % 2 == 0 and nwin >= 4

    @functools.partial(
        pl.kernel, mesh=mesh,
        out_type=jax.ShapeDtypeStruct((nslot, NPAD, width), jnp.float32),
        scratch_types=[pltpu.VMEM((2, WIN), jnp.int32),
                       pltpu.VMEM((2, WIN), jnp.int32),
                       pltpu.VMEM((2, WIN), jnp.int32),
                       pltpu.VMEM((2, WIN), jnp.int32),
                       pltpu.VMEM((2, 2, WIN), jnp.float32),
                       pltpu.VMEM((2, 2, WIN, HID), jnp.float32),
                       pltpu.VMEM((2, WIN, width), jnp.float32),
                       pltpu.VMEM((WIN, width), jnp.float32),
                       pltpu.SemaphoreType.DMA,
                       pltpu.SemaphoreType.DMA,
                       pltpu.SemaphoreType.DMA,
                       pltpu.SemaphoreType.DMA,
                       pltpu.SemaphoreType.DMA,
                       pltpu.SemaphoreType.DMA,
                       pltpu.VMEM_SHARED((NPAD, width), jnp.float32)],
        compiler_params=_SC_PARAMS,
    )
    def agg(hflat_hbm, src_hbm, dst_hbm, wt_hbm, uout_hbm,
            idx0_v, idx1_v, dst_v, dsts_v, wv, rows_v, prod_v, zero_v,
            s00, s01, s10, s11, sp0, sp1, acc_sh):
        core = lax.axis_index("c")
        sid = lax.axis_index("s")
        sems = ((s00, s01), (s10, s11))
        psems = (sp0, sp1)

        @pl.loop(0, WIN)
        def _(j):
            for q in range(width // 16):
                zero_v[j, pl.ds(q * 16, 16)] = jnp.zeros((16,), jnp.float32)

        @pl.loop(0, npass)
        def _(kk):
            if npass > 1:
                k0 = (npass * core + kk) * nh
                slot = npass * core + kk
                ebase = sid * per_w
            else:
                k0 = 0
                slot = core
                ebase = (core * NS + sid) * per_w

            for t in range(NPAD // NS // WIN):
                pltpu.sync_copy(
                    zero_v, acc_sh.at[pl.ds(sid * (NPAD // NS) + t * WIN,
                                            WIN)])
            plsc.subcore_barrier()

            def prefetch(n, b):
                base = ebase + n * WIN
                pltpu.sync_copy(src_hbm.at[pl.ds(base, WIN)], idx0_v.at[b])
                pltpu.sync_copy(dst_hbm.at[pl.ds(base, WIN)], dst_v.at[b])
                pltpu.sync_copy(wt_hbm.at[k0, pl.ds(base, WIN)],
                                wv.at[b, 0])
                if nh > 1:
                    pltpu.sync_copy(wt_hbm.at[k0 + 1, pl.ds(base, WIN)],
                                    wv.at[b, 1])
                for q in range(WIN // 16):
                    sl = pl.ds(q * 16, 16)
                    if nh > 1:
                        idx1_v[b, sl] = idx0_v[b, sl] + (k0 + 1) * NPAD
                    if npass > 1:
                        idx0_v[b, sl] = idx0_v[b, sl] + k0 * NPAD
                pltpu.make_async_copy(hflat_hbm.at[idx0_v.at[b]],
                                      rows_v.at[b, 0], sems[b][0]).start()
                if nh > 1:
                    pltpu.make_async_copy(hflat_hbm.at[idx1_v.at[b]],
                                          rows_v.at[b, 1], sems[b][1]).start()

            def add_copy(b):
                return pltpu.make_async_copy(
                    prod_v.at[b], acc_sh.at[dsts_v.at[b]], psems[b])

            def consume(b):
                pltpu.make_async_copy(hflat_hbm.at[idx0_v.at[b]],
                                      rows_v.at[b, 0], sems[b][0]).wait()
                if nh > 1:
                    pltpu.make_async_copy(hflat_hbm.at[idx1_v.at[b]],
                                          rows_v.at[b, 1], sems[b][1]).wait()
                add_copy(b).wait()
                for q in range(WIN // 16):
                    sl = pl.ds(q * 16, 16)
                    dsts_v[b, sl] = dst_v[b, sl]

                @pl.loop(0, WIN, step=16)
                def _(j0):
                    for jj in range(16):
                        for i in range(nh):
                            splat = jnp.full((16,), 1.0, jnp.float32) * (
                                wv[b, i, pl.ds(j0, 16)][jj])
                            for q in range(HID // 16):
                                prod_v[b, j0 + jj,
                                       pl.ds(i * HID + q * 16, 16)] = (
                                    rows_v[b, i, j0 + jj, pl.ds(q * 16, 16)]
                                    * splat)

                add_copy(b).start(add=True)

            prefetch(0, 0)
            prefetch(1, 1)
            for bb in range(2):
                for q in range(WIN // 16):
                    sl = pl.ds(q * 16, 16)
                    dsts_v[bb, sl] = dst_v[bb, sl]
            pltpu.make_async_copy(zero_v, acc_sh.at[dsts_v.at[0]],
                                  psems[0]).start(add=True)
            pltpu.make_async_copy(zero_v, acc_sh.at[dsts_v.at[1]],
                                  psems[1]).start(add=True)

            @pl.loop(0, nwin // 2 - 1)
            def _(p):
                n = p * 2
                consume(0)
                prefetch(n + 2, 0)
                consume(1)
                prefetch(n + 3, 1)

            consume(0)
            consume(1)
            add_copy(0).wait()
            add_copy(1).wait()

            plsc.subcore_barrier()
            pltpu.sync_copy(
                acc_sh.at[pl.ds(sid * (NPAD // NS), NPAD // NS)],
                uout_hbm.at[slot, pl.ds(sid * (NPAD // NS), NPAD // NS)])
            plsc.subcore_barrier()

    return agg


def _dense2_body(uout_ref, d0_ref, d1_ref, bg1_ref, wg2_ref, as2_ref,
                 ad2_ref, h2_ref, es2_ref, ed2_ref):
    den16 = d0_ref[0] + d1_ref[0] + 1e-16
    parts = []
    for k in range(HEADS):
        num = uout_ref[k]
        den = den16[:, k:k + 1]
        parts.append(jnp.maximum(num / den + bg1_ref[:, k * HID:(k + 1) * HID],
                                 0.0))
    xg = jnp.concatenate(parts, axis=1)
    h2 = _bf16dot(xg, wg2_ref[...])
    h2_ref[...] = h2
    z = jnp.zeros((BR, 15), jnp.float32)
    es2 = jnp.sum(h2 * as2_ref[...], axis=1, keepdims=True)
    ed2 = jnp.sum(h2 * ad2_ref[...], axis=1, keepdims=True)
    es2_ref[...] = jnp.concatenate([es2, z], axis=1)
    ed2_ref[...] = jnp.concatenate([ed2, z], axis=1)


def _final_body(x1_ref, u0_ref, u1_ref, d0_ref, d1_ref, bg2_ref, wp_ref,
                bp_ref, final_ref, g_ref, gt_ref, sq_ref):
    i = pl.program_id(0)
    num = u0_ref[0] + u1_ref[0]
    den = d0_ref[0][:, 0:1] + d1_ref[0][:, 0:1] + 1e-16
    x2 = num / den + bg2_ref[...]
    fb = jnp.concatenate([x1_ref[...], x2], axis=1)
    final_ref[...] = fb
    g = _bf16dot(fb, wp_ref[...]) + bp_ref[...]
    sq = jnp.sum(g * g, axis=1, keepdims=True)
    row = i * BR + lax.broadcasted_iota(jnp.int32, (BR, 1), 0)
    sq_ref[...] = jnp.where(row < N, sq, 1e30)
    g_ref[...] = g
    gt_ref[...] = g.T


BK = 512
NBK = NPAD // BK


def _knn_body(g_ref, gt_ref, sq_ref, sqrow_ref, final_ref, fblk_ref,
              gln_ref, bln_ref, wo1_ref, wo2_ref, bo_ref,
              out_ref, key_ref, mask_ref):
    dots = _bf16dot(g_ref[...], gt_ref[...])
    d = jnp.maximum(sq_ref[...] + sqrow_ref[...] - 2.0 * dots, 0.0)
    key_ref[...] = lax.bitcast_convert_type(d, jnp.int32)

    lo0 = jnp.zeros((BK, 1), jnp.int32)
    hi0 = jnp.full((BK, 1), 0x7F800000, jnp.int32)

    def bs_step(_, carry):
        lo, hi = carry
        mid = lo + ((hi - lo) >> 1)
        cnt = jnp.sum((key_ref[...] <= mid).astype(jnp.int32), axis=1,
                      keepdims=True)
        pred = cnt >= K
        return jnp.where(pred, lo, mid + 1), jnp.where(pred, mid, hi)

    lo, hi = lax.fori_loop(0, 31, bs_step, (lo0, hi0))

    mask = (key_ref[...] <= hi).astype(jnp.float32)
    cnt = jnp.sum(mask, axis=1, keepdims=True)
    mask_ref[...] = mask

    sim = lax.dot_general(mask_ref[...], final_ref[...],
                          (((1,), (0,)), ((), ())), precision=_HI) / cnt
    mu = jnp.mean(sim, axis=1, keepdims=True)
    var = jnp.mean((sim - mu) ** 2, axis=1, keepdims=True)
    simn = (sim - mu) * lax.rsqrt(var + 1e-5) * gln_ref[...] + bln_ref[...]

    out = (lax.dot_general(fblk_ref[...], wo1_ref[...],
                           (((1,), (0,)), ((), ())), precision=_HI)
           + lax.dot_general(simn, wo2_ref[...],
                             (((1,), (0,)), ((), ())), precision=_HI)
           + bo_ref[...])
    out_ref[...] = out


_gw = _make_gw()
_agg1 = _make_agg(4, EPAD, HID)
_agg2 = _make_agg(1, EPAD // 2, HID)


def _full(x, edge_index, W1, b1, Wg1, as1, ad1, bg1, Wg2, as2, ad2, bg2,
          Wp, bp, g_ln, b_ln, Wo, bo):
    xpad = jnp.zeros((NPAD, F_IN), jnp.float32).at[:N].set(x)
    srcp = jnp.zeros((EPAD,), jnp.int32).at[:E].set(edge_index[0])
    dstp = jnp.zeros((EPAD,), jnp.int32).at[:E].set(edge_index[1])

    x1, h, es16, ed16 = pl.pallas_call(
        _dense1_body,
        grid=(NBLK,),
        in_specs=[
            pl.BlockSpec((BR, F_IN), lambda i: (i, 0)),
            pl.BlockSpec((F_IN, HID), lambda i: (0, 0)),
            pl.BlockSpec((1, HID), lambda i: (0, 0)),
            pl.BlockSpec((F_IN, HEADS * HID), lambda i: (0, 0)),
            pl.BlockSpec((HEADS, HID), lambda i: (0, 0)),
            pl.BlockSpec((HEADS, HID), lambda i: (0, 0)),
        ],
        out_specs=[
            pl.BlockSpec((BR, HID), lambda i: (i, 0)),
            pl.BlockSpec((HEADS, BR, HID), lambda i: (0, i, 0)),
            pl.BlockSpec((BR, 16), lambda i: (i, 0)),
            pl.BlockSpec((BR, 16), lambda i: (i, 0)),
        ],
        out_shape=[
            jax.ShapeDtypeStruct((NPAD, HID), jnp.float32),
            jax.ShapeDtypeStruct((HEADS, NPAD, HID), jnp.float32),
            jax.ShapeDtypeStruct((NPAD, 16), jnp.float32),
            jax.ShapeDtypeStruct((NPAD, 16), jnp.float32),
        ],
    )(xpad, W1, b1.reshape(1, HID), Wg1, as1, ad1)

    w16_1, den1 = _gw(es16, ed16, srcp, dstp)

    wt1 = pl.pallas_call(
        _w1_body,
        grid=(NEB,),
        in_specs=[pl.BlockSpec((EBLK, 16), lambda i: (i, 0))],
        out_specs=pl.BlockSpec((HEADS, EBLK), lambda i: (0, i)),
        out_shape=jax.ShapeDtypeStruct((HEADS, EPAD), jnp.float32),
    )(w16_1)

    hflat = h.reshape(HEADS * NPAD, HID)
    uout1 = _agg1(hflat, srcp, dstp, wt1)

    h2, es2, ed2 = pl.pallas_call(
        _dense2_body,
        grid=(NBLK,),
        in_specs=[
            pl.BlockSpec((HEADS, BR, HID), lambda i: (0, i, 0)),
            pl.BlockSpec((1, BR, 16), lambda i: (0, i, 0)),
            pl.BlockSpec((1, BR, 16), lambda i: (1, i, 0)),
            pl.BlockSpec((1, HEADS * HID), lambda i: (0, 0)),
            pl.BlockSpec((HEADS * HID, HID), lambda i: (0, 0)),
            pl.BlockSpec((1, HID), lambda i: (0, 0)),
            pl.BlockSpec((1, HID), lambda i: (0, 0)),
        ],
        out_specs=[pl.BlockSpec((BR, HID), lambda i: (i, 0)),
                   pl.BlockSpec((BR, 16), lambda i: (i, 0)),
                   pl.BlockSpec((BR, 16), lambda i: (i, 0))],
        out_shape=[jax.ShapeDtypeStruct((NPAD, HID), jnp.float32),
                   jax.ShapeDtypeStruct((NPAD, 16), jnp.float32),
                   jax.ShapeDtypeStruct((NPAD, 16), jnp.float32)],
    )(uout1, den1, den1, bg1.reshape(1, -1), Wg2, as2, ad2)

    w16_2, den2 = _gw(es2, ed2, srcp, dstp)

    wt2 = pl.pallas_call(
        _w2_body,
        grid=(NEB,),
        in_specs=[pl.BlockSpec((EBLK, 16), lambda i: (i, 0))],
        out_specs=pl.BlockSpec((1, EBLK), lambda i: (0, i)),
        out_shape=jax.ShapeDtypeStruct((1, EPAD), jnp.float32),
    )(w16_2)

    uout2 = _agg2(h2, srcp, dstp, wt2)

    final, g, gt, sq = pl.pallas_call(
        _final_body,
        grid=(NBLK,),
        in_specs=[
            pl.BlockSpec((BR, HID), lambda i: (i, 0)),
            pl.BlockSpec((1, BR, HID), lambda i: (0, i, 0)),
            pl.BlockSpec((1, BR, HID), lambda i: (1, i, 0)),
            pl.BlockSpec((1, BR, 16), lambda i: (0, i, 0)),
            pl.BlockSpec((1, BR, 16), lambda i: (1, i, 0)),
            pl.BlockSpec((1, HID), lambda i: (0, 0)),
            pl.BlockSpec((2 * HID, PROJ), lambda i: (0, 0)),
            pl.BlockSpec((1, PROJ), lambda i: (0, 0)),
        ],
        out_specs=[
            pl.BlockSpec((BR, 2 * HID), lambda i: (i, 0)),
            pl.BlockSpec((BR, PROJ), lambda i: (i, 0)),
            pl.BlockSpec((PROJ, BR), lambda i: (0, i)),
            pl.BlockSpec((BR, 1), lambda i: (i, 0)),
        ],
        out_shape=[
            jax.ShapeDtypeStruct((NPAD, 2 * HID), jnp.float32),
            jax.ShapeDtypeStruct((NPAD, PROJ), jnp.float32),
            jax.ShapeDtypeStruct((PROJ, NPAD), jnp.float32),
            jax.ShapeDtypeStruct((NPAD, 1), jnp.float32),
        ],
    )(x1, uout2, uout2, den2, den2, bg2.reshape(1, HID), Wp,
      bp.reshape(1, PROJ))

    sqrow = sq.reshape(1, NPAD)

    out = pl.pallas_call(
        _knn_body,
        grid=(NBK,),
        in_specs=[
            pl.BlockSpec((BK, PROJ), lambda i: (i, 0)),
            pl.BlockSpec((PROJ, NPAD), lambda i: (0, 0)),
            pl.BlockSpec((BK, 1), lambda i: (i, 0)),
            pl.BlockSpec((1, NPAD), lambda i: (0, 0)),
            pl.BlockSpec((NPAD, 2 * HID), lambda i: (0, 0)),
            pl.BlockSpec((BK, 2 * HID), lambda i: (i, 0)),
            pl.BlockSpec((1, 2 * HID), lambda i: (0, 0)),
            pl.BlockSpec((1, 2 * HID), lambda i: (0, 0)),
            pl.BlockSpec((2 * HID, HID), lambda i: (0, 0)),
            pl.BlockSpec((2 * HID, HID), lambda i: (0, 0)),
            pl.BlockSpec((1, HID), lambda i: (0, 0)),
        ],
        out_specs=pl.BlockSpec((BK, HID), lambda i: (i, 0)),
        out_shape=jax.ShapeDtypeStruct((NPAD, HID), jnp.float32),
        scratch_shapes=[
            pltpu.VMEM((BK, NPAD), jnp.int32),
            pltpu.VMEM((BK, NPAD), jnp.float32),
        ],
    )(g, gt, sq, sqrow, final, final, g_ln.reshape(1, -1),
      b_ln.reshape(1, -1), Wo[:2 * HID], Wo[2 * HID:], bo.reshape(1, HID))

    return out[:N]


def kernel(x, edge_index, W1, b1, Wg1, as1, ad1, bg1, Wg2, as2, ad2, bg2,
           Wp, bp, g_ln, b_ln, Wo, bo):
    return _full(x, edge_index, W1, b1, Wg1, as1, ad1, bg1, Wg2, as2, ad2,
                 bg2, Wp, bp, g_ln, b_ln, Wo, bo)
